# baseline (speedup 1.0000x reference)
import functools

import jax
import jax.numpy as jnp
from jax import lax
from jax.experimental import pallas as pl
from jax.experimental.pallas import tpu as pltpu
from jax.experimental.pallas import tpu_sc as plsc

_NUM_TABLE = 8
_TABLE_SIZE = 64
_CODE_LEN = 6
_P = 4
_TP = _NUM_TABLE * _P
_ROWS = _NUM_TABLE * _TABLE_SIZE

_B, _H, _S, _D = 2, 16, 2048, 64
_BH = _B * _H
_NCH = 8
_CH = _S // _NCH
_L = 16


def _codes_rows(sT):
    T = _NUM_TABLE
    N = sT.shape[-1]
    planes = [sT[8 * c:8 * (c + 1), :] for c in range(_CODE_LEN)]
    absp = [jnp.abs(p) for p in planes]
    base = jnp.zeros((T, N), jnp.int32)
    for c in range(_CODE_LEN):
        base = base + (planes[c] > 0).astype(jnp.int32) * (1 << c)
    ranks = []
    for c in range(_CODE_LEN):
        r = jnp.zeros((T, N), jnp.int32)
        for c2 in range(_CODE_LEN):
            if c2 == c:
                continue
            lt = absp[c2] < absp[c]
            if c2 < c:
                lt = lt | (absp[c2] == absp[c])
            r = r + lt.astype(jnp.int32)
        ranks.append(r)
    flips = []
    for i in range(_P - 1):
        f = jnp.zeros((T, N), jnp.int32)
        for c in range(_CODE_LEN):
            f = f + (ranks[c] == i).astype(jnp.int32) * (1 << c)
        flips.append(f)
    toff = jax.lax.broadcasted_iota(jnp.int32, (T, N), 0) * _TABLE_SIZE
    rows = [base + toff]
    for i in range(_P - 1):
        rows.append(jnp.bitwise_xor(base, flips[i]) + toff)
    return rows


def _codes_body(q_ref, k_ref, w_ref, qr_ref, kr_ref):
    q = q_ref[0, 0]
    k = k_ref[0, 0]
    w = w_ref[0]
    dn_nt = (((1,), (1,)), ((), ()))
    sq = jax.lax.dot_general(w, q, dn_nt)
    sk = jax.lax.dot_general(w, k, dn_nt)
    qr_ref[0, 0] = jnp.concatenate(_codes_rows(sq), axis=0)
    kr_ref[0, 0] = jnp.concatenate(_codes_rows(sk), axis=0)


_sc_mesh = plsc.VectorSubcoreMesh(core_axis_name="c", subcore_axis_name="s")


@functools.partial(
    pl.kernel,
    out_type=jax.ShapeDtypeStruct((_BH, _NCH, _CH * _D), jnp.float32),
    mesh=_sc_mesh,
    scratch_types=[
        pltpu.VMEM((_ROWS * _D,), jnp.float32),
        pltpu.VMEM((_ROWS,), jnp.float32),
        pltpu.VMEM((_CH * _D,), jnp.float32),
        pltpu.VMEM((_TP, _CH), jnp.int32),
        pltpu.VMEM((_CH * _D,), jnp.float32),
    ],
    compiler_params=pltpu.CompilerParams(needs_layout_passes=False),
)
def _sc_tables(v_hbm, qr_hbm, kr_hbm, out_hbm,
               tab, cnts, vbuf, tpbuf, obuf):
    bh = lax.axis_index("c") * 16 + lax.axis_index("s")
    zero16 = jnp.zeros((_L,), jnp.float32)
    ones16 = jnp.ones((_L,), jnp.float32)
    nsub = _D // _L

    def _zero(ref, n):
        def zb(j, _):
            ref[pl.ds(j * _L, _L)] = zero16
            return 0
        lax.fori_loop(0, n // _L, zb, 0, unroll=4)

    _zero(tab, _ROWS * _D)
    _zero(cnts, _ROWS)

    def build_chunk(ch, _):
        pltpu.sync_copy(v_hbm.at[bh, ch], vbuf)
        pltpu.sync_copy(qr_hbm.at[bh, :, pl.ds(ch * _CH, _CH)], tpbuf)

        def g_loop(g, _):
            gbase = g * _L

            def tp_loop(tp, _):
                rv = tpbuf[tp, pl.ds(gbase, _L)]
                plsc.addupdate_scatter(cnts, [rv], ones16)
                rv64 = rv * _D
                for nn in range(_L):
                    base = rv64[nn]
                    voff = (gbase + nn) * _D
                    for j in range(nsub):
                        vvj = vbuf[pl.ds(voff + j * _L, _L)]
                        plsc.addupdate(tab.at[pl.ds(base + j * _L, _L)], vvj)
                return 0

            return lax.fori_loop(0, _TP, tp_loop, 0)

        return lax.fori_loop(0, _CH // _L, g_loop, 0)

    lax.fori_loop(0, _NCH, build_chunk, 0)

    def query_chunk(ch, _):
        pltpu.sync_copy(kr_hbm.at[bh, :, pl.ds(ch * _CH, _CH)], tpbuf)
        _zero(obuf, _CH * _D)

        def g_loop(g, _):
            gbase = g * _L

            def tp_loop(tp, cacc):
                rv = tpbuf[tp, pl.ds(gbase, _L)]
                cacc = cacc + plsc.load_gather(cnts, [rv])
                rv64 = rv * _D
                for nn in range(_L):
                    base = rv64[nn]
                    ooff = (gbase + nn) * _D
                    for j in range(nsub):
                        val = tab[pl.ds(base + j * _L, _L)]
                        plsc.addupdate(obuf.at[pl.ds(ooff + j * _L, _L)], val)
                return cacc

            cacc = lax.fori_loop(0, _TP, tp_loop, zero16)
            recip = 1.0 / jnp.maximum(cacc, 1.0)
            for nn in range(_L):
                r = recip[nn]
                ooff = (gbase + nn) * _D
                for j in range(nsub):
                    off = ooff + j * _L
                    obuf[pl.ds(off, _L)] = obuf[pl.ds(off, _L)] * r
            return 0

        lax.fori_loop(0, _CH // _L, g_loop, 0)
        pltpu.sync_copy(obuf, out_hbm.at[bh, ch])
        return 0

    lax.fori_loop(0, _NCH, query_chunk, 0)


def kernel(query_layer, key_layer, value_layer, attention_mask, projections):
    B, H, S, D = query_layer.shape
    v = value_layer * attention_mask[:, None, :, None]
    w6 = projections.transpose(0, 2, 1, 3).reshape(H, _CODE_LEN * _NUM_TABLE, D)
    qr, kr = pl.pallas_call(
        _codes_body,
        grid=(B, H),
        in_specs=[
            pl.BlockSpec((1, 1, S, D), lambda b, h: (b, h, 0, 0)),
            pl.BlockSpec((1, 1, S, D), lambda b, h: (b, h, 0, 0)),
            pl.BlockSpec((1, _CODE_LEN * _NUM_TABLE, D), lambda b, h: (h, 0, 0)),
        ],
        out_specs=[
            pl.BlockSpec((1, 1, _TP, S), lambda b, h: (b, h, 0, 0)),
            pl.BlockSpec((1, 1, _TP, S), lambda b, h: (b, h, 0, 0)),
        ],
        out_shape=[
            jax.ShapeDtypeStruct((B, H, _TP, S), jnp.int32),
            jax.ShapeDtypeStruct((B, H, _TP, S), jnp.int32),
        ],
    )(query_layer, key_layer, w6)
    BH = B * H
    qr3 = qr.reshape(BH, _TP, S)
    kr3 = kr.reshape(BH, _TP, S)
    v3 = v.reshape(BH, _NCH, _CH * D)
    out4 = _sc_tables(v3, qr3, kr3)
    return out4.reshape(B, H, S, D)
